# baseline (speedup 1.0000x reference)
import functools

import jax
import jax.numpy as jnp
from jax import lax
from jax.experimental import pallas as pl
from jax.experimental.pallas import tpu as pltpu
from jax.experimental.pallas import tpu_sc as plsc

_VOCAB = 100000
_EMBED = 32
_BATCH = 1024
_CTX = 20

_NC = 2
_NS = 16
_NW = _NC * _NS
_B_PER_W = _BATCH // _NW
_IDX_PER_W = _B_PER_W * _CTX
_IDX_CHUNK = 128
_N_CHUNKS = _IDX_PER_W // _IDX_CHUNK
_LANES = 16


def _sc_gather_mean(idx_grouped, emb_table):
    mesh = plsc.VectorSubcoreMesh(core_axis_name="c", subcore_axis_name="s")

    @functools.partial(
        pl.kernel,
        mesh=mesh,
        out_type=jax.ShapeDtypeStruct((_BATCH, _EMBED), jnp.float32),
        compiler_params=pltpu.CompilerParams(use_tc_tiling_on_sc=False),
        scratch_types=[
            pltpu.VMEM((_N_CHUNKS, _IDX_CHUNK), jnp.int32),
            pltpu.VMEM((_IDX_PER_W, _EMBED), jnp.float32),
            pltpu.VMEM((_B_PER_W, _EMBED), jnp.float32),
            pltpu.SemaphoreType.DMA,
        ],
    )
    def body(idx_hbm, table_hbm, x_hbm, idx_v, rows_v, out_v, sem):
        wid = lax.axis_index("s") * _NC + lax.axis_index("c")
        pltpu.sync_copy(idx_hbm.at[wid], idx_v)
        copies = [
            pltpu.async_copy(
                table_hbm.at[idx_v.at[j]],
                rows_v.at[pl.ds(j * _IDX_CHUNK, _IDX_CHUNK)],
                sem,
            )
            for j in range(_N_CHUNKS)
        ]
        for c in copies:
            c.wait()

        scale = jnp.float32(1.0 / _CTX)

        def accum(i, carry):
            base = i * _CTX
            acc0 = rows_v[base, pl.ds(0, _LANES)]
            acc1 = rows_v[base, pl.ds(_LANES, _LANES)]
            for j in range(1, _CTX):
                acc0 = acc0 + rows_v[base + j, pl.ds(0, _LANES)]
                acc1 = acc1 + rows_v[base + j, pl.ds(_LANES, _LANES)]
            out_v[i, pl.ds(0, _LANES)] = acc0 * scale
            out_v[i, pl.ds(_LANES, _LANES)] = acc1 * scale
            return carry

        lax.fori_loop(0, _B_PER_W, accum, 0)
        pltpu.sync_copy(out_v, x_hbm.at[pl.ds(wid * _B_PER_W, _B_PER_W)])

    return body(idx_grouped, emb_table)


_TILE_V = 2048
_N_STEPS = pl.cdiv(_VOCAB, _TILE_V)
_TAIL_V = 1792
_N_SLOTS = 4
_N_STRIPES = 4
_ROWS = _BATCH // _N_STRIPES


def _mm_body(x_ref, w_ref, b_ref, o_hbm, obuf, osem):
    i = pl.program_id(0)
    slot = lax.rem(i, _N_SLOTS)

    def _stripe_copy(s, step, slot_, width):
        return pltpu.make_async_copy(
            obuf.at[slot_, pl.ds(s * _ROWS, _ROWS), pl.ds(0, width)],
            o_hbm.at[pl.ds(s * _ROWS, _ROWS), pl.ds(step * _TILE_V, width)],
            osem.at[slot_, s],
        )

    @pl.when(i >= _N_SLOTS)
    def _():
        for s in range(_N_STRIPES):
            _stripe_copy(s, i - _N_SLOTS, slot, _TILE_V).wait()

    obuf[slot] = (
        lax.dot_general(
            x_ref[...], w_ref[...],
            (((1,), (1,)), ((), ())),
            preferred_element_type=jnp.float32,
        )
        + b_ref[...]
    )

    @pl.when(i < _N_STEPS - 1)
    def _():
        for s in range(_N_STRIPES):
            _stripe_copy(s, i, slot, _TILE_V).start()

    @pl.when(i == _N_STEPS - 1)
    def _():
        for s in range(_N_STRIPES):
            _stripe_copy(s, i, slot, _TAIL_V).start()
        for back in range(_N_SLOTS - 1, 0, -1):
            for s in range(_N_STRIPES):
                _stripe_copy(s, i - back, lax.rem(i - back, _N_SLOTS),
                             _TILE_V).wait()
        for s in range(_N_STRIPES):
            _stripe_copy(s, i, slot, _TAIL_V).wait()


def _tc_project(x, W, b2d):
    return pl.pallas_call(
        _mm_body,
        grid=(_N_STEPS,),
        in_specs=[
            pl.BlockSpec((_BATCH, _EMBED), lambda i: (0, 0)),
            pl.BlockSpec((_TILE_V, _EMBED), lambda i: (i, 0)),
            pl.BlockSpec((1, _TILE_V), lambda i: (0, i)),
        ],
        out_specs=pl.BlockSpec(memory_space=pltpu.HBM),
        out_shape=jax.ShapeDtypeStruct((_BATCH, _VOCAB), jnp.float32),
        scratch_shapes=[
            pltpu.VMEM((_N_SLOTS, _BATCH, _TILE_V), jnp.float32),
            pltpu.SemaphoreType.DMA((_N_SLOTS, _N_STRIPES)),
        ],
    )(x, W, b2d)


def kernel(inputs, emb_table, W, b):
    x = jnp.mean(jnp.take(emb_table, inputs, axis=0), axis=1)
    return _tc_project(x, W, b.reshape(1, _VOCAB))


# speedup vs baseline: 2.3202x; 2.3202x over previous
import functools

import jax
import jax.numpy as jnp
from jax import lax
from jax.experimental import pallas as pl
from jax.experimental.pallas import tpu as pltpu
from jax.experimental.pallas import tpu_sc as plsc

_VOCAB = 100000
_EMBED = 32
_BATCH = 1024
_CTX = 20

_NC = 2
_NS = 16
_NW = _NC * _NS
_B_PER_W = _BATCH // _NW
_IDX_PER_W = _B_PER_W * _CTX
_IDX_CHUNK = 128
_N_CHUNKS = _IDX_PER_W // _IDX_CHUNK
_LANES = 16


def _sc_gather_mean(idx_grouped, emb_table):
    mesh = plsc.VectorSubcoreMesh(core_axis_name="c", subcore_axis_name="s")

    @functools.partial(
        pl.kernel,
        mesh=mesh,
        out_type=jax.ShapeDtypeStruct((_BATCH, _EMBED), jnp.float32),
        compiler_params=pltpu.CompilerParams(use_tc_tiling_on_sc=False),
        scratch_types=[
            pltpu.VMEM((_N_CHUNKS, _IDX_CHUNK), jnp.int32),
            pltpu.VMEM((_IDX_PER_W, _EMBED), jnp.float32),
            pltpu.VMEM((_B_PER_W, _EMBED), jnp.float32),
            pltpu.SemaphoreType.DMA,
        ],
    )
    def body(idx_hbm, table_hbm, x_hbm, idx_v, rows_v, out_v, sem):
        wid = lax.axis_index("s") * _NC + lax.axis_index("c")
        pltpu.sync_copy(idx_hbm.at[wid], idx_v)
        copies = [
            pltpu.async_copy(
                table_hbm.at[idx_v.at[j]],
                rows_v.at[pl.ds(j * _IDX_CHUNK, _IDX_CHUNK)],
                sem,
            )
            for j in range(_N_CHUNKS)
        ]
        for c in copies:
            c.wait()

        scale = jnp.float32(1.0 / _CTX)

        def accum(i, carry):
            base = i * _CTX
            acc0 = rows_v[base, pl.ds(0, _LANES)]
            acc1 = rows_v[base, pl.ds(_LANES, _LANES)]
            for j in range(1, _CTX):
                acc0 = acc0 + rows_v[base + j, pl.ds(0, _LANES)]
                acc1 = acc1 + rows_v[base + j, pl.ds(_LANES, _LANES)]
            out_v[i, pl.ds(0, _LANES)] = acc0 * scale
            out_v[i, pl.ds(_LANES, _LANES)] = acc1 * scale
            return carry

        lax.fori_loop(0, _B_PER_W, accum, 0)
        pltpu.sync_copy(out_v, x_hbm.at[pl.ds(wid * _B_PER_W, _B_PER_W)])

    return body(idx_grouped, emb_table)


_TILE_V = 2048


def _mm_body(x_ref, wt_ref, b_ref, o_ref):
    o_ref[...] = (
        lax.dot_general(
            wt_ref[...], x_ref[...],
            (((0,), (1,)), ((), ())),
            preferred_element_type=jnp.float32,
        )
        + b_ref[...]
    )


def _tc_project(x, Wt, bcol):
    out_t = pl.pallas_call(
        _mm_body,
        grid=(pl.cdiv(_VOCAB, _TILE_V),),
        in_specs=[
            pl.BlockSpec((_BATCH, _EMBED), lambda i: (0, 0)),
            pl.BlockSpec((_EMBED, _TILE_V), lambda i: (0, i)),
            pl.BlockSpec((_TILE_V, 1), lambda i: (i, 0)),
        ],
        out_specs=pl.BlockSpec((_TILE_V, _BATCH), lambda i: (i, 0)),
        out_shape=jax.ShapeDtypeStruct((_VOCAB, _BATCH), jnp.float32),
    )(x, Wt, bcol)
    return out_t.T


def kernel(inputs, emb_table, W, b):
    idx_grouped = inputs.astype(jnp.int32).reshape(_NW, _N_CHUNKS, _IDX_CHUNK)
    x = _sc_gather_mean(idx_grouped, emb_table)
    return _tc_project(x, W.T, b.reshape(_VOCAB, 1))


# speedup vs baseline: 2.8656x; 1.2351x over previous
import functools

import jax
import jax.numpy as jnp
from jax import lax
from jax.experimental import pallas as pl
from jax.experimental.pallas import tpu as pltpu
from jax.experimental.pallas import tpu_sc as plsc

_VOCAB = 100000
_EMBED = 32
_BATCH = 1024
_CTX = 20

_NC = 2
_NS = 16
_NW = _NC * _NS
_B_PER_W = _BATCH // _NW
_IDX_PER_W = _B_PER_W * _CTX
_IDX_CHUNK = 128
_N_CHUNKS = _IDX_PER_W // _IDX_CHUNK
_LANES = 16


def _sc_gather_mean(idx_grouped, emb_table):
    mesh = plsc.VectorSubcoreMesh(core_axis_name="c", subcore_axis_name="s")

    @functools.partial(
        pl.kernel,
        mesh=mesh,
        out_type=jax.ShapeDtypeStruct((_BATCH, _EMBED), jnp.float32),
        compiler_params=pltpu.CompilerParams(use_tc_tiling_on_sc=False),
        scratch_types=[
            pltpu.VMEM((_N_CHUNKS, _IDX_CHUNK), jnp.int32),
            pltpu.VMEM((_IDX_PER_W, _EMBED), jnp.float32),
            pltpu.VMEM((_B_PER_W, _EMBED), jnp.float32),
            pltpu.SemaphoreType.DMA,
        ],
    )
    def body(idx_hbm, table_hbm, x_hbm, idx_v, rows_v, out_v, sem):
        wid = lax.axis_index("s") * _NC + lax.axis_index("c")
        pltpu.sync_copy(idx_hbm.at[wid], idx_v)
        copies = [
            pltpu.async_copy(
                table_hbm.at[idx_v.at[j]],
                rows_v.at[pl.ds(j * _IDX_CHUNK, _IDX_CHUNK)],
                sem,
            )
            for j in range(_N_CHUNKS)
        ]
        for c in copies:
            c.wait()

        scale = jnp.float32(1.0 / _CTX)

        def accum(i, carry):
            base = i * _CTX
            acc0 = rows_v[base, pl.ds(0, _LANES)]
            acc1 = rows_v[base, pl.ds(_LANES, _LANES)]
            for j in range(1, _CTX):
                acc0 = acc0 + rows_v[base + j, pl.ds(0, _LANES)]
                acc1 = acc1 + rows_v[base + j, pl.ds(_LANES, _LANES)]
            out_v[i, pl.ds(0, _LANES)] = acc0 * scale
            out_v[i, pl.ds(_LANES, _LANES)] = acc1 * scale
            return carry

        lax.fori_loop(0, _B_PER_W, accum, 0)
        pltpu.sync_copy(out_v, x_hbm.at[pl.ds(wid * _B_PER_W, _B_PER_W)])

    return body(idx_grouped, emb_table)


_TILE_V = 2048


def _mm_body(x_ref, wt_ref, o_ref):
    o_ref[...] = lax.dot_general(
        wt_ref[...], x_ref[...],
        (((0,), (1,)), ((), ())),
        preferred_element_type=jnp.float32,
    )


def _tc_project(x1, Wb):
    k = x1.shape[1]
    out_t = pl.pallas_call(
        _mm_body,
        grid=(pl.cdiv(_VOCAB, _TILE_V),),
        in_specs=[
            pl.BlockSpec((_BATCH, k), lambda i: (0, 0)),
            pl.BlockSpec((k, _TILE_V), lambda i: (0, i)),
        ],
        out_specs=pl.BlockSpec((_TILE_V, _BATCH), lambda i: (i, 0)),
        out_shape=jax.ShapeDtypeStruct((_VOCAB, _BATCH), jnp.float32),
    )(x1, Wb)
    return out_t.T


def kernel(inputs, emb_table, W, b):
    idx_grouped = inputs.astype(jnp.int32).reshape(_NW, _N_CHUNKS, _IDX_CHUNK)
    Wb = jnp.concatenate([W.T, b[None, :]], axis=0)
    x = _sc_gather_mean(idx_grouped, emb_table)
    x1 = jnp.concatenate([x, jnp.ones((_BATCH, 1), jnp.float32)], axis=1)
    return _tc_project(x1, Wb)
